# baseline (speedup 1.0000x reference)
import functools

import jax
import jax.numpy as jnp
from jax import lax
from jax.experimental import pallas as pl
from jax.experimental.pallas import tpu as pltpu
from jax.experimental.pallas import tpu_sc as plsc

B = 4096
V = 200
L = 50
D = 64
N = B * L
NC = 2
NS = 16
NW = NC * NS
PW = N // NW
CH = 128
NCH = PW // CH
IPC = CH // 16


def _sc_gather(arr_hbm, idx_hbm, out_hbm, idx_v, buf_v, gsem, osem):
    wid = lax.axis_index("s") * NC + lax.axis_index("c")
    row0 = wid * PW

    pltpu.sync_copy(idx_hbm.at[wid], idx_v)

    lane = lax.broadcasted_iota(jnp.int32, (16,), 0)

    def ibody(t, _):
        j = t // IPC
        k = t - j * IPC
        raw = idx_v[j, pl.ds(k * 16, 16)]
        local = t * 16 + lane
        b = lax.shift_right_logical(local * 10486, 19)
        idx_v[j, pl.ds(k * 16, 16)] = (wid * (PW // L) + b) * V + raw
        return 0

    lax.fori_loop(0, NCH * IPC, ibody, 0)

    def gbody(j, _):
        pltpu.async_copy(arr_hbm.at[idx_v.at[j]], buf_v, gsem).wait()
        pltpu.async_copy(buf_v, out_hbm.at[pl.ds(row0 + j * CH, CH)], osem).wait()
        return 0

    lax.fori_loop(0, NCH, gbody, 0)


@jax.jit
def _run(arr2, idx3):
    mesh = plsc.VectorSubcoreMesh(core_axis_name="c", subcore_axis_name="s")
    f = functools.partial(
        pl.kernel,
        mesh=mesh,
        out_type=jax.ShapeDtypeStruct((N, D), jnp.float32),
        scratch_types=[
            pltpu.VMEM((NCH, CH), jnp.int32),
            pltpu.VMEM((CH, D), jnp.float32),
            pltpu.SemaphoreType.DMA,
            pltpu.SemaphoreType.DMA,
        ],
        compiler_params=pltpu.CompilerParams(use_tc_tiling_on_sc=False),
    )(_sc_gather)
    return f(arr2, idx3)


def kernel(array, indices):
    arr2 = array.reshape(B * V, D)
    idx3 = indices[..., 0].astype(jnp.int32).reshape(NW, NCH, CH)
    out = _run(arr2, idx3)
    return out.reshape(B, L, D)


# speedup vs baseline: 1.0553x; 1.0553x over previous
import functools

import jax
import jax.numpy as jnp
from jax import lax
from jax.experimental import pallas as pl
from jax.experimental.pallas import tpu as pltpu
from jax.experimental.pallas import tpu_sc as plsc

B = 4096
V = 200
L = 50
D = 64
N = B * L
NC = 2
NS = 16
NW = NC * NS
PW = N // NW
CH = 128
NCH = PW // CH
IPC = CH // 16


NBUF = 5
LOOK = 2


def _sc_gather(arr_hbm, idx_hbm, out_hbm, idx_v, bufs, *sems):
    gsems = sems[:NBUF]
    osems = sems[NBUF:]
    wid = lax.axis_index("s") * NC + lax.axis_index("c")
    row0 = wid * PW

    pltpu.sync_copy(idx_hbm.at[wid], idx_v)

    lane = lax.broadcasted_iota(jnp.int32, (16,), 0)

    def ibody(t, _):
        j = t // IPC
        k = t - j * IPC
        raw = idx_v[j, pl.ds(k * 16, 16)]
        local = t * 16 + lane
        b = lax.shift_right_logical(local * 10486, 19)
        idx_v[j, pl.ds(k * 16, 16)] = (wid * (PW // L) + b) * V + raw
        return 0

    lax.fori_loop(0, NCH * IPC, ibody, 0)

    def gather_start(jc, s):
        pltpu.async_copy(arr_hbm.at[idx_v.at[jc]], bufs.at[s], gsems[s])

    def gather_wait(s):
        pltpu.make_async_copy(arr_hbm.at[idx_v.at[0]], bufs.at[s], gsems[s]).wait()

    def out_start(jc, s):
        pltpu.async_copy(bufs.at[s], out_hbm.at[pl.ds(row0 + jc * CH, CH)], osems[s])

    def out_wait(s):
        pltpu.make_async_copy(bufs.at[s], out_hbm.at[pl.ds(0, CH)], osems[s]).wait()

    for j in range(LOOK):
        gather_start(j, j)

    def blk(ib, _):
        j0 = ib * NBUF
        for b in range(NBUF):
            j = j0 + b
            jn = j + LOOK
            sn = (b + LOOK) % NBUF

            @pl.when(jn < NCH)
            def _start_next():
                @pl.when(jn >= NBUF)
                def _drain_out():
                    out_wait(sn)

                gather_start(jn, sn)

            gather_wait(b)
            out_start(j, b)
        return 0

    lax.fori_loop(0, NCH // NBUF, blk, 0)

    for b in range(NBUF):
        out_wait(b)


@jax.jit
def _run(arr2, idx3):
    mesh = plsc.VectorSubcoreMesh(core_axis_name="c", subcore_axis_name="s")
    f = functools.partial(
        pl.kernel,
        mesh=mesh,
        out_type=jax.ShapeDtypeStruct((N, D), jnp.float32),
        scratch_types=[
            pltpu.VMEM((NCH, CH), jnp.int32),
            pltpu.VMEM((NBUF, CH, D), jnp.float32),
        ] + [pltpu.SemaphoreType.DMA] * (2 * NBUF),
        compiler_params=pltpu.CompilerParams(use_tc_tiling_on_sc=False),
    )(_sc_gather)
    return f(arr2, idx3)


def kernel(array, indices):
    arr2 = array.reshape(B * V, D)
    idx3 = indices[..., 0].astype(jnp.int32).reshape(NW, NCH, CH)
    out = _run(arr2, idx3)
    return out.reshape(B, L, D)


# speedup vs baseline: 1.0577x; 1.0023x over previous
import functools

import jax
import jax.numpy as jnp
from jax import lax
from jax.experimental import pallas as pl
from jax.experimental.pallas import tpu as pltpu
from jax.experimental.pallas import tpu_sc as plsc

B = 4096
V = 200
L = 50
D = 64
N = B * L
NC = 2
NS = 16
NW = NC * NS
PW = N // NW
CH = 128
NCH = PW // CH
IPC = CH // 16


NBUF = 5
LOOK = 4


def _sc_gather(arr_hbm, idx_hbm, out_hbm, idx_v, bufs, *sems):
    gsems = sems[:NBUF]
    osems = sems[NBUF:]
    wid = lax.axis_index("s") * NC + lax.axis_index("c")
    row0 = wid * PW

    pltpu.sync_copy(idx_hbm.at[wid], idx_v)

    lane = lax.broadcasted_iota(jnp.int32, (16,), 0)

    def ibody(t, _):
        j = t // IPC
        k = t - j * IPC
        raw = idx_v[j, pl.ds(k * 16, 16)]
        local = t * 16 + lane
        b = lax.shift_right_logical(local * 10486, 19)
        idx_v[j, pl.ds(k * 16, 16)] = (wid * (PW // L) + b) * V + raw
        return 0

    lax.fori_loop(0, NCH * IPC, ibody, 0)

    def gather_start(jc, s):
        pltpu.async_copy(arr_hbm.at[idx_v.at[jc]], bufs.at[s], gsems[s])

    def gather_wait(s):
        pltpu.make_async_copy(arr_hbm.at[idx_v.at[0]], bufs.at[s], gsems[s]).wait()

    def out_start(jc, s):
        pltpu.async_copy(bufs.at[s], out_hbm.at[pl.ds(row0 + jc * CH, CH)], osems[s])

    def out_wait(s):
        pltpu.make_async_copy(bufs.at[s], out_hbm.at[pl.ds(0, CH)], osems[s]).wait()

    for j in range(LOOK):
        gather_start(j, j)

    def blk(ib, _):
        j0 = ib * NBUF
        for b in range(NBUF):
            j = j0 + b
            jn = j + LOOK
            sn = (b + LOOK) % NBUF

            @pl.when(jn < NCH)
            def _start_next():
                @pl.when(jn >= NBUF)
                def _drain_out():
                    out_wait(sn)

                gather_start(jn, sn)

            gather_wait(b)
            out_start(j, b)
        return 0

    lax.fori_loop(0, NCH // NBUF, blk, 0)

    for b in range(NBUF):
        out_wait(b)


@jax.jit
def _run(arr2, idx3):
    mesh = plsc.VectorSubcoreMesh(core_axis_name="c", subcore_axis_name="s")
    f = functools.partial(
        pl.kernel,
        mesh=mesh,
        out_type=jax.ShapeDtypeStruct((N, D), jnp.float32),
        scratch_types=[
            pltpu.VMEM((NCH, CH), jnp.int32),
            pltpu.VMEM((NBUF, CH, D), jnp.float32),
        ] + [pltpu.SemaphoreType.DMA] * (2 * NBUF),
        compiler_params=pltpu.CompilerParams(use_tc_tiling_on_sc=False),
    )(_sc_gather)
    return f(arr2, idx3)


def kernel(array, indices):
    arr2 = array.reshape(B * V, D)
    idx3 = indices[..., 0].astype(jnp.int32).reshape(NW, NCH, CH)
    out = _run(arr2, idx3)
    return out.reshape(B, L, D)


# speedup vs baseline: 4.6830x; 4.4277x over previous
import functools

import jax
import jax.numpy as jnp
from jax import lax
from jax.experimental import pallas as pl
from jax.experimental.pallas import tpu as pltpu
from jax.experimental.pallas import tpu_sc as plsc

B = 4096
V = 200
L = 50
D = 64
NC = 2
NS = 16
NW = NC * NS
BL = B // NW


def _sc_gather(at_hbm, idx_hbm, out_hbm, idx_v, data0, data1, out0, out1, *sems):
    datas = (data0, data1)
    outs = (out0, out1)
    dsems = sems[:2]
    osems = sems[2:]
    wid = lax.axis_index("s") * NC + lax.axis_index("c")
    b0 = wid * BL

    pltpu.sync_copy(idx_hbm.at[:, pl.ds(b0, BL)], idx_v)

    lanes = lax.broadcasted_iota(jnp.int32, (16,), 0)

    def data_start(d, s):
        pltpu.async_copy(at_hbm.at[:, d, pl.ds(b0, BL)], datas[s], dsems[s])

    def data_wait(s):
        pltpu.make_async_copy(
            at_hbm.at[:, 0, pl.ds(b0, BL)], datas[s], dsems[s]
        ).wait()

    def out_start(d, s):
        pltpu.async_copy(outs[s], out_hbm.at[:, d, pl.ds(b0, BL)], osems[s])

    def out_wait(s):
        pltpu.make_async_copy(
            outs[s], out_hbm.at[:, 0, pl.ds(b0, BL)], osems[s]
        ).wait()

    data_start(0, 0)
    data_start(1, 1)

    def lbody(s):
        def body(l, _):
            for g in range(BL // 16):
                v_vec = idx_v[l, pl.ds(g * 16, 16)]
                got = plsc.load_gather(datas[s], [v_vec, lanes + g * 16])
                outs[s][l, pl.ds(g * 16, 16)] = got
            return 0

        lax.fori_loop(0, L, body, 0)

    for d in range(D):
        s = d % 2
        data_wait(s)
        if d >= 2:
            out_wait(s)
        lbody(s)
        out_start(d, s)
        if d + 2 < D:
            data_start(d + 2, s)

    out_wait(0)
    out_wait(1)


@jax.jit
def _run(at, idx2):
    mesh = plsc.VectorSubcoreMesh(core_axis_name="c", subcore_axis_name="s")
    f = functools.partial(
        pl.kernel,
        mesh=mesh,
        out_type=jax.ShapeDtypeStruct((L, D, B), jnp.float32),
        scratch_types=[
            pltpu.VMEM((L, BL), jnp.int32),
            pltpu.VMEM((V, BL), jnp.float32),
            pltpu.VMEM((V, BL), jnp.float32),
            pltpu.VMEM((L, BL), jnp.float32),
            pltpu.VMEM((L, BL), jnp.float32),
        ] + [pltpu.SemaphoreType.DMA] * 4,
        compiler_params=pltpu.CompilerParams(
            use_tc_tiling_on_sc=True, needs_layout_passes=False
        ),
    )(_sc_gather)
    return f(at, idx2)


def kernel(array, indices):
    at = jnp.transpose(array, (1, 2, 0))
    idx2 = indices[..., 0].astype(jnp.int32).T
    out_t = _run(at, idx2)
    return jnp.transpose(out_t, (2, 0, 1))
